# Initial kernel scaffold; baseline (speedup 1.0000x reference)
import jax
import jax.numpy as jnp
from jax.experimental import pallas as pl


def kernel(adj, x, M, W):
    raise NotImplementedError("write your pallas kernel here")


# speedup vs baseline: 3.2158x; 3.2158x over previous
import functools

import jax
import jax.numpy as jnp
from jax.experimental import pallas as pl
from jax.experimental.pallas import tpu as pltpu


def _tgc_kernel(adj_ref, x_ref, M_ref, W_ref, out_ref, gall_ref, q_ref,
                *, T, N, F_IN, F_OUT, BW):
    i = pl.program_id(0)
    s = pl.program_id(1)

    @pl.when(jnp.logical_and(i == 0, s == 0))
    def _init_gall():
        for t in range(T):
            xt = M_ref[t, 0] * x_ref[0]
            for tau in range(1, T):
                xt = xt + M_ref[t, tau] * x_ref[tau]
            g = jax.lax.dot(xt, W_ref[...],
                            precision=jax.lax.Precision.HIGHEST,
                            preferred_element_type=jnp.float32)
            gall_ref[:, t * F_OUT:(t + 1) * F_OUT] = g

    a = adj_ref[0]
    p = jax.lax.dot(a, gall_ref[...],
                    preferred_element_type=jnp.float32)

    gid = jax.lax.broadcasted_iota(jnp.int32, (1, T * F_OUT), 1) // F_OUT
    cvec = jnp.zeros((1, T * F_OUT), jnp.float32)
    for j in range(BW):
        t = s + j
        w = jnp.where(t < T, M_ref[jnp.minimum(t, T - 1), s], 0.0)
        cvec = cvec + jnp.where(gid == t, w, 0.0)
    contrib = p * cvec

    @pl.when(s == 0)
    def _():
        q_ref[...] = contrib

    @pl.when(s > 0)
    def _():
        q_ref[...] = q_ref[...] + contrib

    @pl.when(s == T - 1)
    def _finalize():
        q = q_ref[...]
        for t in range(T):
            out_ref[t] = q[:, t * F_OUT:(t + 1) * F_OUT]


@jax.jit
def kernel(adj, x, M, W):
    T, N, _ = adj.shape
    F_IN = x.shape[2]
    F_OUT = W.shape[1]
    BW = 3
    bN = 512
    body = functools.partial(_tgc_kernel, T=T, N=N, F_IN=F_IN, F_OUT=F_OUT,
                             BW=BW)
    return pl.pallas_call(
        body,
        grid=(N // bN, T),
        in_specs=[
            pl.BlockSpec((1, bN, N), lambda i, s: (s, i, 0)),
            pl.BlockSpec((T, N, F_IN), lambda i, s: (0, 0, 0)),
            pl.BlockSpec(memory_space=pltpu.SMEM),
            pl.BlockSpec((F_IN, F_OUT), lambda i, s: (0, 0)),
        ],
        out_specs=pl.BlockSpec((T, bN, F_OUT), lambda i, s: (0, i, 0)),
        out_shape=jax.ShapeDtypeStruct((T, N, F_OUT), jnp.float32),
        scratch_shapes=[
            pltpu.VMEM((N, T * F_OUT), jnp.float32),
            pltpu.VMEM((bN, T * F_OUT), jnp.float32),
        ],
    )(adj, x, M, W)
